# baseline (speedup 1.0000x reference)
import functools

import jax
import jax.numpy as jnp
from jax import lax
from jax.experimental import pallas as pl
from jax.experimental.pallas import tpu as pltpu
from jax.experimental.pallas import tpu_sc as plsc


@functools.partial(jax.jit, static_argnames=("mode",))
def _sc_aggregate(x, src, dst, attr, mode):
    N, D = x.shape
    E = src.shape[0]
    info = plsc.get_sparse_core_info()
    NC, NS, L = info.num_cores, info.num_subcores, info.num_lanes
    NW = NC * NS
    EW = E // NW
    B = 80
    NB = EW // B
    RW = (N // NS) // 8 * 8
    TAIL = N - NS * RW
    assert EW * NW == E and NB * B == EW and 0 <= TAIL <= B and NB >= 8
    CD = D // L
    is_sum = mode == "sum"
    MAIN = (NB - 3) // 2 * 2

    mesh = plsc.VectorSubcoreMesh(core_axis_name="c", subcore_axis_name="s")

    scratch = [
        pltpu.VMEM((B,), jnp.int32),
        pltpu.VMEM((B,), jnp.int32),
        pltpu.VMEM((B, D), jnp.float32),
        pltpu.VMEM_SHARED((N, D), jnp.float32),
        pltpu.SemaphoreType.DMA,
        pltpu.SemaphoreType.DMA,
        pltpu.SemaphoreType.DMA,
        pltpu.SemaphoreType.DMA,
    ]
    if is_sum:
        scratch += [
            pltpu.VMEM((B,), jnp.int32),
            pltpu.VMEM((B,), jnp.int32),
            pltpu.VMEM((B,), jnp.float32),
            pltpu.VMEM((B,), jnp.float32),
            pltpu.VMEM((B, D), jnp.float32),
            pltpu.VMEM((B, D), jnp.float32),
            pltpu.VMEM((B, D), jnp.float32),
            pltpu.SemaphoreType.DMA,
            pltpu.SemaphoreType.DMA,
        ]

    @functools.partial(
        pl.kernel,
        mesh=mesh,
        out_type=jax.ShapeDtypeStruct((NC, N, D), jnp.float32),
        scratch_types=tuple(scratch),
    )
    def agg(x_hbm, src_hbm, dst_hbm, attr_hbm, out_sums, *rest):
        if is_sum:
            (d0, d1, rc0, accum, i0, i1, s0, s1,
             f0, f1, a0, a1, rc1, rg0, rg1, g0, g1) = rest
            dst_b, src_b, attr_b = (d0, d1), (f0, f1), (a0, a1)
            rowsc, rowsg = (rc0, rc1), (rg0, rg1)
            isem, ssem, gsem = (i0, i1), (s0, s1), (g0, g1)
        else:
            (d0, d1, rc0, accum, i0, i1, s0, s1) = rest
            dst_b = (d0, d1)
            rowsc = (rc0, rc0)
            isem, ssem = (i0, i1), (s0, s1)

        cid = lax.axis_index("c")
        sid = lax.axis_index("s")
        wid = cid * NS + sid
        base_e = wid * EW

        zero16 = jnp.zeros((L,), jnp.float32)
        one16 = jnp.ones((L,), jnp.float32)

        def zrow(r, _):
            for c in range(CD):
                rc0[r, pl.ds(c * L, L)] = zero16
            return 0

        lax.fori_loop(0, B, zrow, 0)

        zsem = isem[0]
        for k in range(RW // B):
            pltpu.async_copy(rc0, accum.at[pl.ds(sid * RW + k * B, B)], zsem)
        rem = RW % B
        if rem:
            pltpu.async_copy(rc0.at[pl.ds(0, rem)],
                             accum.at[pl.ds(sid * RW + (RW // B) * B, rem)],
                             zsem)
        if TAIL:
            @pl.when(sid == 0)
            def _zero_tail():
                pltpu.async_copy(rc0.at[pl.ds(0, TAIL)],
                                 accum.at[pl.ds(NS * RW, TAIL)], zsem)
        for k in range(RW // B):
            pltpu.make_async_copy(
                rc0, accum.at[pl.ds(sid * RW + k * B, B)], zsem).wait()
        if rem:
            pltpu.make_async_copy(
                rc0.at[pl.ds(0, rem)],
                accum.at[pl.ds(sid * RW + (RW // B) * B, rem)], zsem).wait()
        if TAIL:
            @pl.when(sid == 0)
            def _drain_tail():
                pltpu.make_async_copy(
                    rc0.at[pl.ds(0, TAIL)],
                    accum.at[pl.ds(NS * RW, TAIL)], zsem).wait()

        if not is_sum:
            def orow(r, _):
                for c in range(CD):
                    rc0[r, pl.ds(c * L, L)] = one16
                return 0

            lax.fori_loop(0, B, orow, 0)

        plsc.subcore_barrier()

        inv_e = jnp.float32(1.0 / E)

        def issue_idx(kv, t):
            off = kv * B
            pltpu.async_copy(dst_hbm.at[pl.ds(base_e + off, B)],
                             dst_b[t], isem[t])
            if is_sum:
                pltpu.async_copy(src_hbm.at[pl.ds(base_e + off, B)],
                                 src_b[t], isem[t])
                pltpu.async_copy(attr_hbm.at[pl.ds(base_e + off, B)],
                                 attr_b[t], isem[t])

        def wait_idx(kv, t):
            off = kv * B
            pltpu.make_async_copy(dst_hbm.at[pl.ds(base_e + off, B)],
                                  dst_b[t], isem[t]).wait()
            if is_sum:
                pltpu.make_async_copy(src_hbm.at[pl.ds(base_e + off, B)],
                                      src_b[t], isem[t]).wait()
                pltpu.make_async_copy(attr_hbm.at[pl.ds(base_e + off, B)],
                                      attr_b[t], isem[t]).wait()

        def scale(p, r0):
            @plsc.parallel_loop(0, B // L, unroll=2)
            def sub_body(s):
                a16 = attr_b[r0][pl.ds(s * L, L)]
                w16 = inv_e / a16
                for j in range(L):
                    idx = jnp.full((L,), j, dtype=jnp.int32)
                    wj = lax.gather(
                        w16, idx[:, None],
                        lax.GatherDimensionNumbers(
                            offset_dims=(), collapsed_slice_dims=(0,),
                            start_index_map=(0,)),
                        (1,), mode=lax.GatherScatterMode.PROMISE_IN_BOUNDS)
                    e = s * L + j
                    for c in range(CD):
                        rowsc[p][e, pl.ds(c * L, L)] = (
                            rowsg[p][e, pl.ds(c * L, L)] * wj)

        def do_block(kv, k_static, first=False):
            p = k_static % 2
            q = 1 - p
            issue1 = (k_static + 1 <= NB - 1) if k_static >= MAIN else True

            if not first:
                pltpu.make_async_copy(
                    rowsc[q], accum.at[dst_b[q]], ssem[q]).wait()
            if issue1:
                issue_idx(kv + 1, q)
            if is_sum:
                pltpu.make_async_copy(
                    x_hbm.at[src_b[p]], rowsg[p], gsem[p]).wait()
                scale(p, p)
            if issue1:
                wait_idx(kv + 1, q)
                if is_sum:
                    pltpu.async_copy(x_hbm.at[src_b[q]], rowsg[q], gsem[q])
            pltpu.async_copy(rowsc[p], accum.at[dst_b[p]], ssem[p],
                             add=True)

        issue_idx(0, 0)
        wait_idx(0, 0)
        if is_sum:
            pltpu.async_copy(x_hbm.at[src_b[0]], rowsg[0], gsem[0])
        do_block(0, 0, first=True)

        def main_body(i, _):
            for u in range(2):
                do_block(1 + i * 2 + u, 1 + u)
            return 0

        lax.fori_loop(0, MAIN // 2, main_body, 0)

        for k in range(MAIN + 1, NB):
            do_block(k, k)

        pltpu.make_async_copy(
            rowsc[(NB - 1) % 2], accum.at[dst_b[(NB - 1) % 2]],
            ssem[(NB - 1) % 2]).wait()

        plsc.subcore_barrier()

        pltpu.sync_copy(accum.at[pl.ds(sid * RW, RW)],
                        out_sums.at[cid, pl.ds(sid * RW, RW)])
        if TAIL:
            @pl.when(sid == 0)
            def _write_tail():
                pltpu.sync_copy(accum.at[pl.ds(NS * RW, TAIL)],
                                out_sums.at[cid, pl.ds(NS * RW, TAIL)])

    return agg(x, src, dst, attr)


def _tc_layer(x, sums, cnt, W, b):
    N, D = x.shape
    H = W.shape[0]
    BN = 1000
    b2d = b.reshape(1, H)

    def body(x_ref, p_ref, c_ref, w_ref, b_ref, o_ref):
        xa = x_ref[...]
        s = p_ref[0] + p_ref[1]
        c = (c_ref[0] + c_ref[1])[:, 0:1]
        aggr = s / jnp.maximum(c, 1.0)
        t = lax.dot_general(
            xa - aggr, w_ref[...],
            (((1,), (1,)), ((), ())),
            preferred_element_type=jnp.float32,
        )
        t = t + b_ref[...]
        o_ref[...] = jnp.maximum(t, 0.0)

    return pl.pallas_call(
        body,
        grid=(N // BN,),
        in_specs=[
            pl.BlockSpec((BN, D), lambda i: (i, 0)),
            pl.BlockSpec((2, BN, D), lambda i: (0, i, 0)),
            pl.BlockSpec((2, BN, D), lambda i: (0, i, 0)),
            pl.BlockSpec((H, D), lambda i: (0, 0)),
            pl.BlockSpec((1, H), lambda i: (0, 0)),
        ],
        out_specs=pl.BlockSpec((BN, H), lambda i: (i, 0)),
        out_shape=jax.ShapeDtypeStruct((N, H), jnp.float32),
    )(x, sums, cnt, W, b2d)


def kernel(x, edge_index, edge_attr, W1, b1, W2, b2):
    src = edge_index[0]
    dst = edge_index[1]
    cnt = _sc_aggregate(x, src, dst, edge_attr, "count")
    sums1 = _sc_aggregate(x, src, dst, edge_attr, "sum")
    h1 = _tc_layer(x, sums1, cnt, W1, b1)
    sums2 = _sc_aggregate(h1, src, dst, edge_attr, "sum")
    h2 = _tc_layer(h1, sums2, cnt, W2, b2)
    return h2


# speedup vs baseline: 1.0820x; 1.0820x over previous
import functools

import jax
import jax.numpy as jnp
from jax import lax
from jax.experimental import pallas as pl
from jax.experimental.pallas import tpu as pltpu
from jax.experimental.pallas import tpu_sc as plsc


@functools.partial(jax.jit, static_argnames=("mode",))
def _sc_aggregate(x, src, dst, attr, mode):
    N, D = x.shape
    E = src.shape[0]
    info = plsc.get_sparse_core_info()
    NC, NS, L = info.num_cores, info.num_subcores, info.num_lanes
    NW = NC * NS
    EW = E // NW
    B = 80
    NB = EW // B
    RW = (N // NS) // 8 * 8
    TAIL = N - NS * RW
    assert EW * NW == E and NB * B == EW and 0 <= TAIL <= B and NB >= 8
    CD = D // L
    is_sum = mode == "sum"
    MAIN = (NB - 3) // 2 * 2

    mesh = plsc.VectorSubcoreMesh(core_axis_name="c", subcore_axis_name="s")

    scratch = [
        pltpu.VMEM((B,), jnp.int32),
        pltpu.VMEM((B,), jnp.int32),
        pltpu.VMEM((B, D), jnp.float32),
        pltpu.VMEM_SHARED((N, D), jnp.float32),
        pltpu.SemaphoreType.DMA,
        pltpu.SemaphoreType.DMA,
        pltpu.SemaphoreType.DMA,
        pltpu.SemaphoreType.DMA,
    ]
    if is_sum:
        scratch += [
            pltpu.VMEM((B,), jnp.int32),
            pltpu.VMEM((B,), jnp.int32),
            pltpu.VMEM((B,), jnp.float32),
            pltpu.VMEM((B,), jnp.float32),
            pltpu.VMEM((B, D), jnp.float32),
            pltpu.VMEM((B, D), jnp.float32),
            pltpu.VMEM((B, D), jnp.float32),
            pltpu.SemaphoreType.DMA,
            pltpu.SemaphoreType.DMA,
        ]

    @functools.partial(
        pl.kernel,
        mesh=mesh,
        out_type=jax.ShapeDtypeStruct((NC, N, D), jnp.float32),
        scratch_types=tuple(scratch),
    )
    def agg(x_hbm, src_hbm, dst_hbm, attr_hbm, out_sums, *rest):
        if is_sum:
            (d0, d1, rc0, accum, i0, i1, s0, s1,
             f0, f1, a0, a1, rc1, rg0, rg1, g0, g1) = rest
            dst_b, src_b, attr_b = (d0, d1), (f0, f1), (a0, a1)
            rowsc, rowsg = (rc0, rc1), (rg0, rg1)
            isem, ssem, gsem = (i0, i1), (s0, s1), (g0, g1)
        else:
            (d0, d1, rc0, accum, i0, i1, s0, s1) = rest
            dst_b = (d0, d1)
            rowsc = (rc0, rc0)
            isem, ssem = (i0, i1), (s0, s1)

        cid = lax.axis_index("c")
        sid = lax.axis_index("s")
        wid = cid * NS + sid
        base_e = wid * EW

        zero16 = jnp.zeros((L,), jnp.float32)
        one16 = jnp.ones((L,), jnp.float32)

        def zrow(r, _):
            for c in range(CD):
                rc0[r, pl.ds(c * L, L)] = zero16
            return 0

        lax.fori_loop(0, B, zrow, 0)

        zsem = isem[0]
        for k in range(RW // B):
            pltpu.async_copy(rc0, accum.at[pl.ds(sid * RW + k * B, B)], zsem)
        rem = RW % B
        if rem:
            pltpu.async_copy(rc0.at[pl.ds(0, rem)],
                             accum.at[pl.ds(sid * RW + (RW // B) * B, rem)],
                             zsem)
        if TAIL:
            @pl.when(sid == 0)
            def _zero_tail():
                pltpu.async_copy(rc0.at[pl.ds(0, TAIL)],
                                 accum.at[pl.ds(NS * RW, TAIL)], zsem)
        for k in range(RW // B):
            pltpu.make_async_copy(
                rc0, accum.at[pl.ds(sid * RW + k * B, B)], zsem).wait()
        if rem:
            pltpu.make_async_copy(
                rc0.at[pl.ds(0, rem)],
                accum.at[pl.ds(sid * RW + (RW // B) * B, rem)], zsem).wait()
        if TAIL:
            @pl.when(sid == 0)
            def _drain_tail():
                pltpu.make_async_copy(
                    rc0.at[pl.ds(0, TAIL)],
                    accum.at[pl.ds(NS * RW, TAIL)], zsem).wait()

        if not is_sum:
            def orow(r, _):
                for c in range(CD):
                    rc0[r, pl.ds(c * L, L)] = one16
                return 0

            lax.fori_loop(0, B, orow, 0)

        plsc.subcore_barrier()

        inv_e = jnp.float32(1.0 / E)

        def issue_idx(kv, t):
            off = kv * B
            pltpu.async_copy(dst_hbm.at[pl.ds(base_e + off, B)],
                             dst_b[t], isem[t])
            if is_sum:
                pltpu.async_copy(src_hbm.at[pl.ds(base_e + off, B)],
                                 src_b[t], isem[t])
                pltpu.async_copy(attr_hbm.at[pl.ds(base_e + off, B)],
                                 attr_b[t], isem[t])

        def wait_idx(kv, t):
            off = kv * B
            pltpu.make_async_copy(dst_hbm.at[pl.ds(base_e + off, B)],
                                  dst_b[t], isem[t]).wait()
            if is_sum:
                pltpu.make_async_copy(src_hbm.at[pl.ds(base_e + off, B)],
                                      src_b[t], isem[t]).wait()
                pltpu.make_async_copy(attr_hbm.at[pl.ds(base_e + off, B)],
                                      attr_b[t], isem[t]).wait()

        def scale(p, r0):
            @plsc.parallel_loop(0, B // L, unroll=1)
            def sub_body(s):
                a16 = attr_b[r0][pl.ds(s * L, L)]
                w16 = inv_e / a16
                for j in range(L):
                    idx = jnp.full((L,), j, dtype=jnp.int32)
                    wj = lax.gather(
                        w16, idx[:, None],
                        lax.GatherDimensionNumbers(
                            offset_dims=(), collapsed_slice_dims=(0,),
                            start_index_map=(0,)),
                        (1,), mode=lax.GatherScatterMode.PROMISE_IN_BOUNDS)
                    e = s * L + j
                    for c in range(CD):
                        rowsc[p][e, pl.ds(c * L, L)] = (
                            rowsg[p][e, pl.ds(c * L, L)] * wj)

        def do_block(kv, k_static, first=False):
            p = k_static % 2
            q = 1 - p
            issue1 = (k_static + 1 <= NB - 1) if k_static >= MAIN else True

            if not first:
                pltpu.make_async_copy(
                    rowsc[q], accum.at[dst_b[q]], ssem[q]).wait()
            if issue1:
                issue_idx(kv + 1, q)
            if is_sum:
                pltpu.make_async_copy(
                    x_hbm.at[src_b[p]], rowsg[p], gsem[p]).wait()
                scale(p, p)
            if issue1:
                wait_idx(kv + 1, q)
                if is_sum:
                    pltpu.async_copy(x_hbm.at[src_b[q]], rowsg[q], gsem[q])
            pltpu.async_copy(rowsc[p], accum.at[dst_b[p]], ssem[p],
                             add=True)

        issue_idx(0, 0)
        wait_idx(0, 0)
        if is_sum:
            pltpu.async_copy(x_hbm.at[src_b[0]], rowsg[0], gsem[0])
        do_block(0, 0, first=True)

        def main_body(i, _):
            for u in range(2):
                do_block(1 + i * 2 + u, 1 + u)
            return 0

        lax.fori_loop(0, MAIN // 2, main_body, 0)

        for k in range(MAIN + 1, NB):
            do_block(k, k)

        pltpu.make_async_copy(
            rowsc[(NB - 1) % 2], accum.at[dst_b[(NB - 1) % 2]],
            ssem[(NB - 1) % 2]).wait()

        plsc.subcore_barrier()

        pltpu.sync_copy(accum.at[pl.ds(sid * RW, RW)],
                        out_sums.at[cid, pl.ds(sid * RW, RW)])
        if TAIL:
            @pl.when(sid == 0)
            def _write_tail():
                pltpu.sync_copy(accum.at[pl.ds(NS * RW, TAIL)],
                                out_sums.at[cid, pl.ds(NS * RW, TAIL)])

    return agg(x, src, dst, attr)


def _tc_layer(x, sums, cnt, W, b):
    N, D = x.shape
    H = W.shape[0]
    BN = 1000
    b2d = b.reshape(1, H)

    def body(x_ref, p_ref, c_ref, w_ref, b_ref, o_ref):
        xa = x_ref[...]
        s = p_ref[0] + p_ref[1]
        c = (c_ref[0] + c_ref[1])[:, 0:1]
        aggr = s / jnp.maximum(c, 1.0)
        t = lax.dot_general(
            xa - aggr, w_ref[...],
            (((1,), (1,)), ((), ())),
            preferred_element_type=jnp.float32,
        )
        t = t + b_ref[...]
        o_ref[...] = jnp.maximum(t, 0.0)

    return pl.pallas_call(
        body,
        grid=(N // BN,),
        in_specs=[
            pl.BlockSpec((BN, D), lambda i: (i, 0)),
            pl.BlockSpec((2, BN, D), lambda i: (0, i, 0)),
            pl.BlockSpec((2, BN, D), lambda i: (0, i, 0)),
            pl.BlockSpec((H, D), lambda i: (0, 0)),
            pl.BlockSpec((1, H), lambda i: (0, 0)),
        ],
        out_specs=pl.BlockSpec((BN, H), lambda i: (i, 0)),
        out_shape=jax.ShapeDtypeStruct((N, H), jnp.float32),
    )(x, sums, cnt, W, b2d)


def kernel(x, edge_index, edge_attr, W1, b1, W2, b2):
    src = edge_index[0]
    dst = edge_index[1]
    cnt = _sc_aggregate(x, src, dst, edge_attr, "count")
    sums1 = _sc_aggregate(x, src, dst, edge_attr, "sum")
    h1 = _tc_layer(x, sums1, cnt, W1, b1)
    sums2 = _sc_aggregate(h1, src, dst, edge_attr, "sum")
    h2 = _tc_layer(h1, sums2, cnt, W2, b2)
    return h2


# speedup vs baseline: 1.1742x; 1.0852x over previous
import functools

import jax
import jax.numpy as jnp
from jax import lax
from jax.experimental import pallas as pl
from jax.experimental.pallas import tpu as pltpu
from jax.experimental.pallas import tpu_sc as plsc


@functools.partial(jax.jit, static_argnames=("mode",))
def _sc_aggregate(x, src, dst, attr, mode):
    N, D = x.shape
    E = src.shape[0]
    info = plsc.get_sparse_core_info()
    NC, NS, L = info.num_cores, info.num_subcores, info.num_lanes
    NW = NC * NS
    EW = E // NW
    B = 80
    NB = EW // B
    RW = (N // NS) // 8 * 8
    TAIL = N - NS * RW
    assert EW * NW == E and NB * B == EW and 0 <= TAIL <= B and NB >= 8
    CD = D // L
    is_sum = mode == "sum"
    MAIN = (NB - 3) // 2 * 2

    mesh = plsc.VectorSubcoreMesh(core_axis_name="c", subcore_axis_name="s")

    scratch = [
        pltpu.VMEM((B,), jnp.int32),
        pltpu.VMEM((B,), jnp.int32),
        pltpu.VMEM((B, D), jnp.float32),
        pltpu.VMEM_SHARED((N, D), jnp.float32),
        pltpu.SemaphoreType.DMA,
        pltpu.SemaphoreType.DMA,
        pltpu.SemaphoreType.DMA,
        pltpu.SemaphoreType.DMA,
    ]
    if is_sum:
        scratch += [
            pltpu.VMEM((B,), jnp.int32),
            pltpu.VMEM((B,), jnp.int32),
            pltpu.VMEM((B,), jnp.float32),
            pltpu.VMEM((B,), jnp.float32),
            pltpu.VMEM((B, D), jnp.float32),
            pltpu.VMEM((B, D), jnp.float32),
            pltpu.VMEM((B, D), jnp.float32),
            pltpu.SemaphoreType.DMA,
            pltpu.SemaphoreType.DMA,
        ]

    @functools.partial(
        pl.kernel,
        mesh=mesh,
        out_type=jax.ShapeDtypeStruct((NC, N, D), jnp.float32),
        scratch_types=tuple(scratch),
    )
    def agg(x_hbm, src_hbm, dst_hbm, attr_hbm, out_sums, *rest):
        if is_sum:
            (d0, d1, rc0, accum, i0, i1, s0, s1,
             f0, f1, a0, a1, rc1, rg0, rg1, g0, g1) = rest
            dst_b, src_b, attr_b = (d0, d1), (f0, f1), (a0, a1)
            rowsc, rowsg = (rc0, rc1), (rg0, rg1)
            isem, ssem, gsem = (i0, i1), (s0, s1), (g0, g1)
        else:
            (d0, d1, rc0, accum, i0, i1, s0, s1) = rest
            dst_b = (d0, d1)
            rowsc = (rc0, rc0)
            isem, ssem = (i0, i1), (s0, s1)

        cid = lax.axis_index("c")
        sid = lax.axis_index("s")
        wid = cid * NS + sid
        base_e = wid * EW

        zero16 = jnp.zeros((L,), jnp.float32)
        one16 = jnp.ones((L,), jnp.float32)

        def zrow(r, _):
            for c in range(CD):
                rc0[r, pl.ds(c * L, L)] = zero16
            return 0

        lax.fori_loop(0, B, zrow, 0)

        zsem = isem[0]
        for k in range(RW // B):
            pltpu.async_copy(rc0, accum.at[pl.ds(sid * RW + k * B, B)], zsem)
        rem = RW % B
        if rem:
            pltpu.async_copy(rc0.at[pl.ds(0, rem)],
                             accum.at[pl.ds(sid * RW + (RW // B) * B, rem)],
                             zsem)
        if TAIL:
            @pl.when(sid == 0)
            def _zero_tail():
                pltpu.async_copy(rc0.at[pl.ds(0, TAIL)],
                                 accum.at[pl.ds(NS * RW, TAIL)], zsem)
        for k in range(RW // B):
            pltpu.make_async_copy(
                rc0, accum.at[pl.ds(sid * RW + k * B, B)], zsem).wait()
        if rem:
            pltpu.make_async_copy(
                rc0.at[pl.ds(0, rem)],
                accum.at[pl.ds(sid * RW + (RW // B) * B, rem)], zsem).wait()
        if TAIL:
            @pl.when(sid == 0)
            def _drain_tail():
                pltpu.make_async_copy(
                    rc0.at[pl.ds(0, TAIL)],
                    accum.at[pl.ds(NS * RW, TAIL)], zsem).wait()

        if not is_sum:
            def orow(r, _):
                for c in range(CD):
                    rc0[r, pl.ds(c * L, L)] = one16
                return 0

            lax.fori_loop(0, B, orow, 0)

        plsc.subcore_barrier()

        inv_e = jnp.float32(1.0 / E)

        def issue_idx(kv, t):
            off = kv * B
            pltpu.async_copy(dst_hbm.at[pl.ds(base_e + off, B)],
                             dst_b[t], isem[t])
            if is_sum:
                pltpu.async_copy(src_hbm.at[pl.ds(base_e + off, B)],
                                 src_b[t], isem[t])
                pltpu.async_copy(attr_hbm.at[pl.ds(base_e + off, B)],
                                 attr_b[t], isem[t])

        def wait_idx(kv, t):
            off = kv * B
            pltpu.make_async_copy(dst_hbm.at[pl.ds(base_e + off, B)],
                                  dst_b[t], isem[t]).wait()
            if is_sum:
                pltpu.make_async_copy(src_hbm.at[pl.ds(base_e + off, B)],
                                      src_b[t], isem[t]).wait()
                pltpu.make_async_copy(attr_hbm.at[pl.ds(base_e + off, B)],
                                      attr_b[t], isem[t]).wait()

        def scale(p, r0):
            @plsc.parallel_loop(0, B // L, unroll=1)
            def sub_body(s):
                a16 = attr_b[r0][pl.ds(s * L, L)]
                w16 = inv_e / a16
                for j in range(L):
                    idx = jnp.full((L,), j, dtype=jnp.int32)
                    wj = lax.gather(
                        w16, idx[:, None],
                        lax.GatherDimensionNumbers(
                            offset_dims=(), collapsed_slice_dims=(0,),
                            start_index_map=(0,)),
                        (1,), mode=lax.GatherScatterMode.PROMISE_IN_BOUNDS)
                    e = s * L + j
                    for c in range(CD):
                        rowsc[p][e, pl.ds(c * L, L)] = (
                            rowsg[p][e, pl.ds(c * L, L)] * wj)

        def do_block(kv, k_static, first=False):
            p = k_static % 2
            q = 1 - p
            issue1 = (k_static + 1 <= NB - 1) if k_static >= MAIN else True

            if not first:
                pltpu.make_async_copy(
                    rowsc[q], accum.at[dst_b[q]], ssem[q]).wait()
            if issue1:
                issue_idx(kv + 1, q)
            if is_sum:
                pltpu.make_async_copy(
                    x_hbm.at[src_b[p]], rowsg[p], gsem[p]).wait()
            if issue1:
                wait_idx(kv + 1, q)
                if is_sum:
                    pltpu.async_copy(x_hbm.at[src_b[q]], rowsg[q], gsem[q])
            if is_sum:
                scale(p, p)
            pltpu.async_copy(rowsc[p], accum.at[dst_b[p]], ssem[p],
                             add=True)

        issue_idx(0, 0)
        wait_idx(0, 0)
        if is_sum:
            pltpu.async_copy(x_hbm.at[src_b[0]], rowsg[0], gsem[0])
        do_block(0, 0, first=True)

        def main_body(i, _):
            for u in range(2):
                do_block(1 + i * 2 + u, 1 + u)
            return 0

        lax.fori_loop(0, MAIN // 2, main_body, 0)

        for k in range(MAIN + 1, NB):
            do_block(k, k)

        pltpu.make_async_copy(
            rowsc[(NB - 1) % 2], accum.at[dst_b[(NB - 1) % 2]],
            ssem[(NB - 1) % 2]).wait()

        plsc.subcore_barrier()

        pltpu.sync_copy(accum.at[pl.ds(sid * RW, RW)],
                        out_sums.at[cid, pl.ds(sid * RW, RW)])
        if TAIL:
            @pl.when(sid == 0)
            def _write_tail():
                pltpu.sync_copy(accum.at[pl.ds(NS * RW, TAIL)],
                                out_sums.at[cid, pl.ds(NS * RW, TAIL)])

    return agg(x, src, dst, attr)


def _tc_layer(x, sums, cnt, W, b):
    N, D = x.shape
    H = W.shape[0]
    BN = 1000
    b2d = b.reshape(1, H)

    def body(x_ref, p_ref, c_ref, w_ref, b_ref, o_ref):
        xa = x_ref[...]
        s = p_ref[0] + p_ref[1]
        c = (c_ref[0] + c_ref[1])[:, 0:1]
        aggr = s / jnp.maximum(c, 1.0)
        t = lax.dot_general(
            xa - aggr, w_ref[...],
            (((1,), (1,)), ((), ())),
            preferred_element_type=jnp.float32,
        )
        t = t + b_ref[...]
        o_ref[...] = jnp.maximum(t, 0.0)

    return pl.pallas_call(
        body,
        grid=(N // BN,),
        in_specs=[
            pl.BlockSpec((BN, D), lambda i: (i, 0)),
            pl.BlockSpec((2, BN, D), lambda i: (0, i, 0)),
            pl.BlockSpec((2, BN, D), lambda i: (0, i, 0)),
            pl.BlockSpec((H, D), lambda i: (0, 0)),
            pl.BlockSpec((1, H), lambda i: (0, 0)),
        ],
        out_specs=pl.BlockSpec((BN, H), lambda i: (i, 0)),
        out_shape=jax.ShapeDtypeStruct((N, H), jnp.float32),
    )(x, sums, cnt, W, b2d)


def kernel(x, edge_index, edge_attr, W1, b1, W2, b2):
    src = edge_index[0]
    dst = edge_index[1]
    cnt = _sc_aggregate(x, src, dst, edge_attr, "count")
    sums1 = _sc_aggregate(x, src, dst, edge_attr, "sum")
    h1 = _tc_layer(x, sums1, cnt, W1, b1)
    sums2 = _sc_aggregate(h1, src, dst, edge_attr, "sum")
    h2 = _tc_layer(h1, sums2, cnt, W2, b2)
    return h2


# speedup vs baseline: 1.1838x; 1.0082x over previous
import functools

import jax
import jax.numpy as jnp
from jax import lax
from jax.experimental import pallas as pl
from jax.experimental.pallas import tpu as pltpu
from jax.experimental.pallas import tpu_sc as plsc


@functools.partial(jax.jit, static_argnames=("mode",))
def _sc_aggregate(x, src, dst, attr, mode):
    N, D = x.shape
    E = src.shape[0]
    info = plsc.get_sparse_core_info()
    NC, NS, L = info.num_cores, info.num_subcores, info.num_lanes
    NW = NC * NS
    EW = E // NW
    B = 80
    NB = EW // B
    RW = (N // NS) // 8 * 8
    TAIL = N - NS * RW
    assert EW * NW == E and NB * B == EW and 0 <= TAIL <= B and NB >= 8
    CD = D // L
    is_sum = mode == "sum"
    MAIN = (NB - 3) // 2 * 2

    mesh = plsc.VectorSubcoreMesh(core_axis_name="c", subcore_axis_name="s")

    scratch = [
        pltpu.VMEM((B,), jnp.int32),
        pltpu.VMEM((B,), jnp.int32),
        pltpu.VMEM((B, D), jnp.float32),
        pltpu.VMEM_SHARED((N, D), jnp.float32),
        pltpu.SemaphoreType.DMA,
        pltpu.SemaphoreType.DMA,
        pltpu.SemaphoreType.DMA,
        pltpu.SemaphoreType.DMA,
    ]
    if is_sum:
        scratch += [
            pltpu.VMEM((B,), jnp.int32),
            pltpu.VMEM((B,), jnp.int32),
            pltpu.VMEM((B,), jnp.float32),
            pltpu.VMEM((B,), jnp.float32),
            pltpu.VMEM((B, D), jnp.float32),
            pltpu.VMEM((B, D), jnp.float32),
            pltpu.VMEM((B, D), jnp.float32),
            pltpu.SemaphoreType.DMA,
            pltpu.SemaphoreType.DMA,
        ]

    @functools.partial(
        pl.kernel,
        mesh=mesh,
        out_type=jax.ShapeDtypeStruct((NC, N, D), jnp.float32),
        scratch_types=tuple(scratch),
    )
    def agg(x_hbm, src_hbm, dst_hbm, attr_hbm, out_sums, *rest):
        if is_sum:
            (d0, d1, rc0, accum, i0, i1, s0, s1,
             f0, f1, a0, a1, rc1, rg0, rg1, g0, g1) = rest
            dst_b, src_b, attr_b = (d0, d1), (f0, f1), (a0, a1)
            rowsc, rowsg = (rc0, rc1), (rg0, rg1)
            isem, ssem, gsem = (i0, i1), (s0, s1), (g0, g1)
        else:
            (d0, d1, rc0, accum, i0, i1, s0, s1) = rest
            dst_b = (d0, d1)
            rowsc = (rc0, rc0)
            isem, ssem = (i0, i1), (s0, s1)

        cid = lax.axis_index("c")
        sid = lax.axis_index("s")
        wid = cid * NS + sid
        base_e = wid * EW

        zero16 = jnp.zeros((L,), jnp.float32)
        one16 = jnp.ones((L,), jnp.float32)

        def zrow(r, _):
            for c in range(CD):
                rc0[r, pl.ds(c * L, L)] = zero16
            return 0

        lax.fori_loop(0, B, zrow, 0)

        zsem = isem[0]
        for k in range(RW // B):
            pltpu.async_copy(rc0, accum.at[pl.ds(sid * RW + k * B, B)], zsem)
        rem = RW % B
        if rem:
            pltpu.async_copy(rc0.at[pl.ds(0, rem)],
                             accum.at[pl.ds(sid * RW + (RW // B) * B, rem)],
                             zsem)
        if TAIL:
            @pl.when(sid == 0)
            def _zero_tail():
                pltpu.async_copy(rc0.at[pl.ds(0, TAIL)],
                                 accum.at[pl.ds(NS * RW, TAIL)], zsem)
        for k in range(RW // B):
            pltpu.make_async_copy(
                rc0, accum.at[pl.ds(sid * RW + k * B, B)], zsem).wait()
        if rem:
            pltpu.make_async_copy(
                rc0.at[pl.ds(0, rem)],
                accum.at[pl.ds(sid * RW + (RW // B) * B, rem)], zsem).wait()
        if TAIL:
            @pl.when(sid == 0)
            def _drain_tail():
                pltpu.make_async_copy(
                    rc0.at[pl.ds(0, TAIL)],
                    accum.at[pl.ds(NS * RW, TAIL)], zsem).wait()

        if not is_sum:
            def orow(r, _):
                for c in range(CD):
                    rc0[r, pl.ds(c * L, L)] = one16
                return 0

            lax.fori_loop(0, B, orow, 0)

        plsc.subcore_barrier()

        inv_e = jnp.float32(1.0 / E)

        def issue_idx(kv, t):
            off = kv * B
            pltpu.async_copy(dst_hbm.at[pl.ds(base_e + off, B)],
                             dst_b[t], isem[t])
            if is_sum:
                pltpu.async_copy(src_hbm.at[pl.ds(base_e + off, B)],
                                 src_b[t], isem[t])
                pltpu.async_copy(attr_hbm.at[pl.ds(base_e + off, B)],
                                 attr_b[t], isem[t])

        def wait_idx(kv, t):
            off = kv * B
            pltpu.make_async_copy(dst_hbm.at[pl.ds(base_e + off, B)],
                                  dst_b[t], isem[t]).wait()
            if is_sum:
                pltpu.make_async_copy(src_hbm.at[pl.ds(base_e + off, B)],
                                      src_b[t], isem[t]).wait()
                pltpu.make_async_copy(attr_hbm.at[pl.ds(base_e + off, B)],
                                      attr_b[t], isem[t]).wait()

        def scale(p, r0):
            @plsc.parallel_loop(0, B // L, unroll=1)
            def sub_body(s):
                a16 = attr_b[r0][pl.ds(s * L, L)]
                w16 = inv_e / a16
                for j in range(L):
                    idx = jnp.full((L,), j, dtype=jnp.int32)
                    wj = lax.gather(
                        w16, idx[:, None],
                        lax.GatherDimensionNumbers(
                            offset_dims=(), collapsed_slice_dims=(0,),
                            start_index_map=(0,)),
                        (1,), mode=lax.GatherScatterMode.PROMISE_IN_BOUNDS)
                    e = s * L + j
                    for c in range(CD):
                        rowsc[p][e, pl.ds(c * L, L)] = (
                            rowsg[p][e, pl.ds(c * L, L)] * wj)

        def do_block(kv, k_static, first=False):
            p = k_static % 2
            q = 1 - p
            issue1 = (k_static + 1 <= NB - 1) if k_static >= MAIN else True

            if not first:
                pltpu.make_async_copy(
                    rowsc[q], accum.at[dst_b[q]], ssem[q]).wait()
            if issue1:
                issue_idx(kv + 1, q)
            if is_sum:
                pltpu.make_async_copy(
                    x_hbm.at[src_b[p]], rowsg[p], gsem[p]).wait()
            if issue1:
                wait_idx(kv + 1, q)
                if is_sum:
                    pltpu.async_copy(x_hbm.at[src_b[q]], rowsg[q], gsem[q])
            if is_sum:
                scale(p, p)
            pltpu.async_copy(rowsc[p], accum.at[dst_b[p]], ssem[p],
                             add=True)

        issue_idx(0, 0)
        wait_idx(0, 0)
        if is_sum:
            pltpu.async_copy(x_hbm.at[src_b[0]], rowsg[0], gsem[0])
        do_block(0, 0, first=True)

        def main_body(i, _):
            for u in range(2):
                do_block(1 + i * 2 + u, 1 + u)
            return 0

        lax.fori_loop(0, MAIN // 2, main_body, 0)

        for k in range(MAIN + 1, NB):
            do_block(k, k)

        pltpu.make_async_copy(
            rowsc[(NB - 1) % 2], accum.at[dst_b[(NB - 1) % 2]],
            ssem[(NB - 1) % 2]).wait()

        plsc.subcore_barrier()

        pltpu.sync_copy(accum.at[pl.ds(sid * RW, RW)],
                        out_sums.at[cid, pl.ds(sid * RW, RW)])
        if TAIL:
            @pl.when(sid == 0)
            def _write_tail():
                pltpu.sync_copy(accum.at[pl.ds(NS * RW, TAIL)],
                                out_sums.at[cid, pl.ds(NS * RW, TAIL)])

    return agg(x, src, dst, attr)


def _tc_layer(x, sums, cnt, W, b):
    N, D = x.shape
    H = W.shape[0]
    BN = 2000
    b2d = b.reshape(1, H)

    def body(x_ref, p_ref, c_ref, w_ref, b_ref, o_ref):
        xa = x_ref[...]
        s = p_ref[0] + p_ref[1]
        c = (c_ref[0] + c_ref[1])[:, 0:1]
        aggr = s / jnp.maximum(c, 1.0)
        t = lax.dot_general(
            xa - aggr, w_ref[...],
            (((1,), (1,)), ((), ())),
            preferred_element_type=jnp.float32,
        )
        t = t + b_ref[...]
        o_ref[...] = jnp.maximum(t, 0.0)

    return pl.pallas_call(
        body,
        grid=(N // BN,),
        in_specs=[
            pl.BlockSpec((BN, D), lambda i: (i, 0)),
            pl.BlockSpec((2, BN, D), lambda i: (0, i, 0)),
            pl.BlockSpec((2, BN, D), lambda i: (0, i, 0)),
            pl.BlockSpec((H, D), lambda i: (0, 0)),
            pl.BlockSpec((1, H), lambda i: (0, 0)),
        ],
        out_specs=pl.BlockSpec((BN, H), lambda i: (i, 0)),
        out_shape=jax.ShapeDtypeStruct((N, H), jnp.float32),
    )(x, sums, cnt, W, b2d)


def kernel(x, edge_index, edge_attr, W1, b1, W2, b2):
    src = edge_index[0]
    dst = edge_index[1]
    cnt = _sc_aggregate(x, src, dst, edge_attr, "count")
    sums1 = _sc_aggregate(x, src, dst, edge_attr, "sum")
    h1 = _tc_layer(x, sums1, cnt, W1, b1)
    sums2 = _sc_aggregate(h1, src, dst, edge_attr, "sum")
    h2 = _tc_layer(h1, sums2, cnt, W2, b2)
    return h2


# speedup vs baseline: 1.2684x; 1.0715x over previous
import functools

import jax
import jax.numpy as jnp
from jax import lax
from jax.experimental import pallas as pl
from jax.experimental.pallas import tpu as pltpu
from jax.experimental.pallas import tpu_sc as plsc


@functools.partial(jax.jit, static_argnames=("mode",))
def _sc_aggregate(x, src, dst, attr, mode):
    N, D = x.shape
    E = src.shape[0]
    info = plsc.get_sparse_core_info()
    NC, NS, L = info.num_cores, info.num_subcores, info.num_lanes
    NW = NC * NS
    EW = E // NW
    B = 80
    NB = EW // B
    RW = (N // NS) // 8 * 8
    TAIL = N - NS * RW
    assert EW * NW == E and NB * B == EW and 0 <= TAIL <= B and NB >= 8
    CD = D // L
    is_sum = mode == "sum"
    MAIN = (NB - 5) // 6 * 6

    mesh = plsc.VectorSubcoreMesh(core_axis_name="c", subcore_axis_name="s")

    scratch = [
        pltpu.VMEM((B,), jnp.int32),
        pltpu.VMEM((B,), jnp.int32),
        pltpu.VMEM((B,), jnp.int32),
        pltpu.VMEM((B, D), jnp.float32),
        pltpu.VMEM_SHARED((N, D), jnp.float32),
        pltpu.SemaphoreType.DMA,
        pltpu.SemaphoreType.DMA,
        pltpu.SemaphoreType.DMA,
        pltpu.SemaphoreType.DMA,
        pltpu.SemaphoreType.DMA,
    ]
    if is_sum:
        scratch += [
            pltpu.VMEM((B,), jnp.int32),
            pltpu.VMEM((B,), jnp.int32),
            pltpu.VMEM((B,), jnp.int32),
            pltpu.VMEM((B,), jnp.float32),
            pltpu.VMEM((B,), jnp.float32),
            pltpu.VMEM((B,), jnp.float32),
            pltpu.VMEM((B, D), jnp.float32),
            pltpu.VMEM((B, D), jnp.float32),
            pltpu.VMEM((B, D), jnp.float32),
            pltpu.SemaphoreType.DMA,
            pltpu.SemaphoreType.DMA,
        ]

    @functools.partial(
        pl.kernel,
        mesh=mesh,
        out_type=jax.ShapeDtypeStruct((NC, N, D), jnp.float32),
        scratch_types=tuple(scratch),
    )
    def agg(x_hbm, src_hbm, dst_hbm, attr_hbm, out_sums, *rest):
        if is_sum:
            (d0, d1, d2, rc0, accum, i0, i1, i2, s0, s1,
             f0, f1, f2, a0, a1, a2, rc1, rg0, rg1, g0, g1) = rest
            dst_b, src_b, attr_b = (d0, d1, d2), (f0, f1, f2), (a0, a1, a2)
            rowsc, rowsg = (rc0, rc1), (rg0, rg1)
            isem, ssem, gsem = (i0, i1, i2), (s0, s1), (g0, g1)
        else:
            (d0, d1, d2, rc0, accum, i0, i1, i2, s0, s1) = rest
            dst_b = (d0, d1, d2)
            rowsc = (rc0, rc0)
            isem, ssem = (i0, i1, i2), (s0, s1)

        cid = lax.axis_index("c")
        sid = lax.axis_index("s")
        wid = cid * NS + sid
        base_e = wid * EW

        zero16 = jnp.zeros((L,), jnp.float32)
        one16 = jnp.ones((L,), jnp.float32)

        def zrow(r, _):
            for c in range(CD):
                rc0[r, pl.ds(c * L, L)] = zero16
            return 0

        lax.fori_loop(0, B, zrow, 0)

        zsem = isem[0]
        for k in range(RW // B):
            pltpu.async_copy(rc0, accum.at[pl.ds(sid * RW + k * B, B)], zsem)
        rem = RW % B
        if rem:
            pltpu.async_copy(rc0.at[pl.ds(0, rem)],
                             accum.at[pl.ds(sid * RW + (RW // B) * B, rem)],
                             zsem)
        if TAIL:
            @pl.when(sid == 0)
            def _zero_tail():
                pltpu.async_copy(rc0.at[pl.ds(0, TAIL)],
                                 accum.at[pl.ds(NS * RW, TAIL)], zsem)
        for k in range(RW // B):
            pltpu.make_async_copy(
                rc0, accum.at[pl.ds(sid * RW + k * B, B)], zsem).wait()
        if rem:
            pltpu.make_async_copy(
                rc0.at[pl.ds(0, rem)],
                accum.at[pl.ds(sid * RW + (RW // B) * B, rem)], zsem).wait()
        if TAIL:
            @pl.when(sid == 0)
            def _drain_tail():
                pltpu.make_async_copy(
                    rc0.at[pl.ds(0, TAIL)],
                    accum.at[pl.ds(NS * RW, TAIL)], zsem).wait()

        if not is_sum:
            def orow(r, _):
                for c in range(CD):
                    rc0[r, pl.ds(c * L, L)] = one16
                return 0

            lax.fori_loop(0, B, orow, 0)

        plsc.subcore_barrier()

        inv_e = jnp.float32(1.0 / E)

        def issue_idx(kv, t):
            off = kv * B
            pltpu.async_copy(dst_hbm.at[pl.ds(base_e + off, B)],
                             dst_b[t], isem[t])
            if is_sum:
                pltpu.async_copy(src_hbm.at[pl.ds(base_e + off, B)],
                                 src_b[t], isem[t])
                pltpu.async_copy(attr_hbm.at[pl.ds(base_e + off, B)],
                                 attr_b[t], isem[t])

        def wait_idx(kv, t):
            off = kv * B
            pltpu.make_async_copy(dst_hbm.at[pl.ds(base_e + off, B)],
                                  dst_b[t], isem[t]).wait()
            if is_sum:
                pltpu.make_async_copy(src_hbm.at[pl.ds(base_e + off, B)],
                                      src_b[t], isem[t]).wait()
                pltpu.make_async_copy(attr_hbm.at[pl.ds(base_e + off, B)],
                                      attr_b[t], isem[t]).wait()

        def scale(p, r0):
            @plsc.parallel_loop(0, B // L, unroll=1)
            def sub_body(s):
                a16 = attr_b[r0][pl.ds(s * L, L)]
                w16 = inv_e / a16
                for j in range(L):
                    idx = jnp.full((L,), j, dtype=jnp.int32)
                    wj = lax.gather(
                        w16, idx[:, None],
                        lax.GatherDimensionNumbers(
                            offset_dims=(), collapsed_slice_dims=(0,),
                            start_index_map=(0,)),
                        (1,), mode=lax.GatherScatterMode.PROMISE_IN_BOUNDS)
                    e = s * L + j
                    for c in range(CD):
                        rowsc[p][e, pl.ds(c * L, L)] = (
                            rowsg[p][e, pl.ds(c * L, L)] * wj)

        def do_block(kv, k_static, first=False):
            p = k_static % 2
            q = 1 - p
            r0 = k_static % 3
            r1 = (k_static + 1) % 3
            r2 = (k_static + 2) % 3
            issue2 = (k_static + 2 <= NB - 1) if k_static >= MAIN else True
            issue1 = (k_static + 1 <= NB - 1) if k_static >= MAIN else True

            if not first:
                pltpu.make_async_copy(
                    rowsc[q], accum.at[dst_b[r2]], ssem[q]).wait()
            if issue2:
                issue_idx(kv + 2, r2)
            if is_sum:
                pltpu.make_async_copy(
                    x_hbm.at[src_b[r0]], rowsg[p], gsem[p]).wait()
            if issue1:
                wait_idx(kv + 1, r1)
                if is_sum:
                    pltpu.async_copy(x_hbm.at[src_b[r1]], rowsg[q], gsem[q])
            if is_sum:
                scale(p, r0)
            pltpu.async_copy(rowsc[p], accum.at[dst_b[r0]], ssem[p],
                             add=True)

        issue_idx(0, 0)
        issue_idx(1, 1)
        wait_idx(0, 0)
        if is_sum:
            pltpu.async_copy(x_hbm.at[src_b[0]], rowsg[0], gsem[0])
        do_block(0, 0, first=True)

        def main_body(i, _):
            for u in range(6):
                do_block(1 + i * 6 + u, 1 + u)
            return 0

        lax.fori_loop(0, MAIN // 6, main_body, 0)

        for k in range(MAIN + 1, NB):
            do_block(k, k)

        pltpu.make_async_copy(
            rowsc[(NB - 1) % 2], accum.at[dst_b[(NB - 1) % 3]],
            ssem[(NB - 1) % 2]).wait()

        plsc.subcore_barrier()

        pltpu.sync_copy(accum.at[pl.ds(sid * RW, RW)],
                        out_sums.at[cid, pl.ds(sid * RW, RW)])
        if TAIL:
            @pl.when(sid == 0)
            def _write_tail():
                pltpu.sync_copy(accum.at[pl.ds(NS * RW, TAIL)],
                                out_sums.at[cid, pl.ds(NS * RW, TAIL)])

    return agg(x, src, dst, attr)


def _tc_layer(x, sums, cnt, W, b):
    N, D = x.shape
    H = W.shape[0]
    BN = 2000
    b2d = b.reshape(1, H)

    def body(x_ref, p_ref, c_ref, w_ref, b_ref, o_ref):
        xa = x_ref[...]
        s = p_ref[0] + p_ref[1]
        c = (c_ref[0] + c_ref[1])[:, 0:1]
        aggr = s / jnp.maximum(c, 1.0)
        t = lax.dot_general(
            xa - aggr, w_ref[...],
            (((1,), (1,)), ((), ())),
            preferred_element_type=jnp.float32,
        )
        t = t + b_ref[...]
        o_ref[...] = jnp.maximum(t, 0.0)

    return pl.pallas_call(
        body,
        grid=(N // BN,),
        in_specs=[
            pl.BlockSpec((BN, D), lambda i: (i, 0)),
            pl.BlockSpec((2, BN, D), lambda i: (0, i, 0)),
            pl.BlockSpec((2, BN, D), lambda i: (0, i, 0)),
            pl.BlockSpec((H, D), lambda i: (0, 0)),
            pl.BlockSpec((1, H), lambda i: (0, 0)),
        ],
        out_specs=pl.BlockSpec((BN, H), lambda i: (i, 0)),
        out_shape=jax.ShapeDtypeStruct((N, H), jnp.float32),
    )(x, sums, cnt, W, b2d)


def kernel(x, edge_index, edge_attr, W1, b1, W2, b2):
    src = edge_index[0]
    dst = edge_index[1]
    cnt = _sc_aggregate(x, src, dst, edge_attr, "count")
    sums1 = _sc_aggregate(x, src, dst, edge_attr, "sum")
    h1 = _tc_layer(x, sums1, cnt, W1, b1)
    sums2 = _sc_aggregate(h1, src, dst, edge_attr, "sum")
    h2 = _tc_layer(h1, sums2, cnt, W2, b2)
    return h2
